# Initial kernel scaffold; baseline (speedup 1.0000x reference)
import jax
import jax.numpy as jnp
from jax.experimental import pallas as pl


def kernel(user_sparse, tables):
    raise NotImplementedError("write your pallas kernel here")


# speedup vs baseline: 12.1582x; 12.1582x over previous
import functools

import jax
import jax.numpy as jnp
from jax import lax
from jax.experimental import pallas as pl
from jax.experimental.pallas import tpu as pltpu
from jax.experimental.pallas import tpu_sc as plsc

_BATCH = 16384
_NF = 26
_NEMB = 13
_VOCAB = 500
_DIM = 4
_OUT = 64
_NW = 32
_CHUNK = _BATCH // _NW
_GROUPS = _CHUNK // 16
_LANES = 16


def _sc_body(tab_hbm, us_hbm, out_hbm, tab_v, us_v, out_v):
    wid = lax.axis_index("s") * 2 + lax.axis_index("c")
    pltpu.sync_copy(tab_hbm, tab_v)
    pltpu.sync_copy(us_hbm.at[pl.ds(wid * (_CHUNK * _NF), _CHUNK * _NF)], us_v)

    def group(g, carry):
        rows = g * _LANES + lax.iota(jnp.int32, _LANES)
        us_base = rows * _NF
        out_base = rows * _OUT
        for f in range(_NEMB):
            idx = plsc.load_gather(us_v, [us_base + f])
            addr = idx * _DIM + (f * _VOCAB * _DIM)
            for d in range(_DIM):
                v = plsc.load_gather(tab_v, [addr + d])
                plsc.store_scatter(out_v, [out_base + (4 * f + d)], v)
        for f in range(_NEMB, _NF - 1):
            iv = plsc.load_gather(us_v, [us_base + f])
            plsc.store_scatter(out_v, [out_base + (f + 39)],
                               iv.astype(jnp.float32))
        return carry

    lax.fori_loop(0, _GROUPS, group, 0)
    pltpu.sync_copy(out_v, out_hbm.at[pl.ds(wid * (_CHUNK * _OUT),
                                            _CHUNK * _OUT)])


@jax.jit
def kernel(user_sparse, tables):
    tab_flat = tables.reshape(-1)
    us_flat = user_sparse.reshape(-1)
    mesh = plsc.VectorSubcoreMesh(core_axis_name="c", subcore_axis_name="s")
    run = functools.partial(
        pl.kernel,
        mesh=mesh,
        compiler_params=pltpu.CompilerParams(needs_layout_passes=False),
        out_type=jax.ShapeDtypeStruct((_BATCH * _OUT,), jnp.float32),
        scratch_types=[
            pltpu.VMEM((_NEMB * _VOCAB * _DIM,), jnp.float32),
            pltpu.VMEM((_CHUNK * _NF,), jnp.int32),
            pltpu.VMEM((_CHUNK * _OUT,), jnp.float32),
        ],
    )(_sc_body)
    return run(tab_flat, us_flat).reshape(_BATCH, _OUT)


# speedup vs baseline: 12.9500x; 1.0651x over previous
import functools

import jax
import jax.numpy as jnp
from jax import lax
from jax.experimental import pallas as pl
from jax.experimental.pallas import tpu as pltpu
from jax.experimental.pallas import tpu_sc as plsc

_BATCH = 16384
_NF = 26
_NEMB = 13
_VOCAB = 500
_DIM = 4
_OUT = 64
_NW = 32
_CHUNK = _BATCH // _NW
_PASS = 256
_NPASS = _CHUNK // _PASS
_GROUPS = _PASS // 16
_LANES = 16


def _sc_body(tab_hbm, us_hbm, out_hbm, tab_v, us_v, out_v):
    wid = lax.axis_index("s") * 2 + lax.axis_index("c")
    pltpu.sync_copy(tab_hbm, tab_v)

    for p in range(_NPASS):
        base = wid * _CHUNK + p * _PASS
        pltpu.sync_copy(us_hbm.at[pl.ds(base, _PASS)], us_v)

        def group(g, carry):
            rows = g * _LANES + lax.iota(jnp.int32, _LANES)
            for f in range(_NEMB):
                fvec = jnp.full((_LANES,), f, jnp.int32)
                idx = plsc.load_gather(us_v, [rows, fvec])
                addr = idx * _DIM + (f * _VOCAB * _DIM)
                for d in range(_DIM):
                    v = plsc.load_gather(tab_v, [addr + d])
                    cvec = jnp.full((_LANES,), 4 * f + d, jnp.int32)
                    plsc.store_scatter(out_v, [rows, cvec], v)
            for f in range(_NEMB, _NF - 1):
                fvec = jnp.full((_LANES,), f, jnp.int32)
                iv = plsc.load_gather(us_v, [rows, fvec])
                cvec = jnp.full((_LANES,), f + 39, jnp.int32)
                plsc.store_scatter(out_v, [rows, cvec], iv.astype(jnp.float32))
            return carry

        lax.fori_loop(0, _GROUPS, group, 0)
        pltpu.sync_copy(out_v, out_hbm.at[pl.ds(base, _PASS)])


@jax.jit
def kernel(user_sparse, tables):
    mesh = plsc.VectorSubcoreMesh(core_axis_name="c", subcore_axis_name="s")
    run = functools.partial(
        pl.kernel,
        mesh=mesh,
        compiler_params=pltpu.CompilerParams(needs_layout_passes=False),
        out_type=jax.ShapeDtypeStruct((_BATCH, _OUT), jnp.float32),
        scratch_types=[
            pltpu.VMEM((_NEMB * _VOCAB * _DIM,), jnp.float32),
            pltpu.VMEM((_PASS, _NF), jnp.int32),
            pltpu.VMEM((_PASS, _OUT), jnp.float32),
        ],
    )(_sc_body)
    return run(tables.reshape(-1), user_sparse)


# speedup vs baseline: 23.5337x; 1.8173x over previous
import functools

import jax
import jax.numpy as jnp
from jax import lax
from jax.experimental import pallas as pl
from jax.experimental.pallas import tpu as pltpu
from jax.experimental.pallas import tpu_sc as plsc

_BATCH = 16384
_NF = 26
_NEMB = 13
_VOCAB = 500
_DIM = 4
_OUT = 64
_NW = 32
_CHUNK = _BATCH // _NW
_PROWS = 128
_NPASS = _CHUNK // _PROWS
_PGROUPS = _PROWS // 16
_LANES = 16


def _sc_body(tab_hbm, us_hbm, out_hbm, tab_v, us_v, out_v0, out_v1,
             sem_t, sem_u, sem_o0, sem_o1):
    wid = lax.axis_index("s") * 2 + lax.axis_index("c")
    chunk0 = wid * _CHUNK
    ct = pltpu.async_copy(tab_hbm, tab_v, sem_t)
    cu = pltpu.async_copy(us_hbm.at[pl.ds(chunk0, _CHUNK)], us_v, sem_u)
    ct.wait()
    cu.wait()

    out_bufs = (out_v0, out_v1)
    out_sems = (sem_o0, sem_o1)
    copies = [None, None]
    for p in range(_NPASS):
        buf = out_bufs[p % 2]

        @functools.partial(plsc.parallel_loop, 0, _PGROUPS, unroll=2)
        def _group(g, buf=buf, p=p):
            lrows = g * _LANES + lax.iota(jnp.int32, _LANES)
            grows = lrows + (p * _PROWS)
            idxs = []
            for f in range(_NF - 1):
                fv = jnp.full((_LANES,), f, jnp.int32)
                idxs.append(plsc.load_gather(us_v, [grows, fv]))
            vals = []
            for f in range(_NEMB):
                addr = idxs[f] * _DIM + (f * _VOCAB * _DIM)
                for d in range(_DIM):
                    vals.append(plsc.load_gather(tab_v, [addr + d]))
            for c in range(_NEMB * _DIM):
                cv = jnp.full((_LANES,), c, jnp.int32)
                plsc.store_scatter(buf, [lrows, cv], vals[c])
            for f in range(_NEMB, _NF - 1):
                cv = jnp.full((_LANES,), f + 39, jnp.int32)
                plsc.store_scatter(buf, [lrows, cv],
                                   idxs[f].astype(jnp.float32))

        pltpu.sync_copy(buf, out_hbm.at[pl.ds(chunk0 + p * _PROWS, _PROWS)])


@jax.jit
def kernel(user_sparse, tables):
    mesh = plsc.VectorSubcoreMesh(core_axis_name="c", subcore_axis_name="s")
    run = functools.partial(
        pl.kernel,
        mesh=mesh,
        compiler_params=pltpu.CompilerParams(needs_layout_passes=False),
        out_type=jax.ShapeDtypeStruct((_BATCH, _OUT), jnp.float32),
        scratch_types=[
            pltpu.VMEM((_NEMB * _VOCAB * _DIM,), jnp.float32),
            pltpu.VMEM((_CHUNK, _NF), jnp.int32),
            pltpu.VMEM((_PROWS, _OUT), jnp.float32),
            pltpu.VMEM((_PROWS, _OUT), jnp.float32),
            pltpu.SemaphoreType.DMA,
            pltpu.SemaphoreType.DMA,
            pltpu.SemaphoreType.DMA,
            pltpu.SemaphoreType.DMA,
        ],
    )(_sc_body)
    return run(tables.reshape(-1), user_sparse)
